# baseline (speedup 1.0000x reference)
import jax
import jax.numpy as jnp
from jax import lax
from jax.experimental import pallas as pl
from jax.experimental.pallas import tpu as pltpu
from jax.experimental.pallas import tpu_sc as plsc

D_MODEL = 2048
NUM_EXPERTS = 8
TOP_K = 2
NUM_TASKS = 64
D_TASK_EMBED = 64
D_FFN = 32
EF = NUM_EXPERTS * D_FFN

TOK_BLK = 512

SC_CORES = 2
SC_SUBCORES = 16
SC_WORKERS = SC_CORES * SC_SUBCORES
SC_LANES = 16


def _stage12_body(tid_ref, x_ref, w1_ref, b1_ref, gw_ref, gb_ref, temb_ref,
                  h_ref, lg_ref, w_ref, stats_ref):
    i = pl.program_id(0)
    nb = pl.num_programs(0) // 2
    blk_per_b = nb // stats_ref.shape[1]
    n_elem = jnp.float32(blk_per_b * TOK_BLK * D_MODEL)
    x = x_ref[...]

    @pl.when(i == 0)
    def _init():
        for b in range(stats_ref.shape[1]):
            stats_ref[0, b] = 0.0
            stats_ref[1, b] = 0.0

    @pl.when(i < nb)
    def _phase_a():
        b = i // blk_per_b
        stats_ref[0, b] += jnp.sum(x)
        stats_ref[1, b] += jnp.sum(x * x)
        xb = x.astype(jnp.bfloat16)
        h = jnp.concatenate(
            [jnp.dot(xb, w1_ref[e], preferred_element_type=jnp.float32)
             for e in range(NUM_EXPERTS)], axis=1)
        h = h + b1_ref[...]
        h_ref[...] = (h * jax.nn.sigmoid(h)).astype(jnp.bfloat16)

    @pl.when(i >= nb)
    def _phase_b():
        b = (i - nb) // blk_per_b
        mean = stats_ref[0, b] / n_elem
        var = stats_ref[1, b] / n_elem - mean * mean
        std = jnp.sqrt(var + 1e-5)
        xn = (x - mean) / std
        tid = tid_ref[b]
        tb = temb_ref[pl.ds(tid, 1), :]
        te_row = jnp.dot(tb.astype(jnp.bfloat16),
                         gw_ref[pl.ds(D_MODEL, D_TASK_EMBED), :].astype(jnp.bfloat16),
                         preferred_element_type=jnp.float32) + gb_ref[...]
        logits = jnp.dot(xn.astype(jnp.bfloat16),
                         gw_ref[pl.ds(0, D_MODEL), :].astype(jnp.bfloat16),
                         preferred_element_type=jnp.float32) + te_row
        lg_ref[...] = logits
        ii = lax.broadcasted_iota(jnp.int32, logits.shape, 1)
        m1 = jnp.max(logits, axis=1, keepdims=True)
        i1 = jnp.min(jnp.where(logits == m1, ii, NUM_EXPERTS), axis=1,
                     keepdims=True)
        l2 = jnp.where(ii == i1, -jnp.inf, logits)
        m2 = jnp.max(l2, axis=1, keepdims=True)
        i2 = jnp.min(jnp.where(l2 == m2, ii, NUM_EXPERTS), axis=1,
                     keepdims=True)
        ed = jnp.exp(m2 - m1)
        p1 = 1.0 / (1.0 + ed)
        p2 = ed / (1.0 + ed)
        w_ref[...] = jnp.where(ii == i1, p1, 0.0) + jnp.where(ii == i2, p2, 0.0)


def _route_body(lg_hbm, idxt_hbm, cnt_hbm, lg_v, i1_v, i2_v, cnt_v):
    tok_per_w = lg_hbm.shape[0] // SC_WORKERS
    n_grp = tok_per_w // SC_LANES
    wid = lax.axis_index("s") * SC_CORES + lax.axis_index("c")
    base = wid * tok_per_w
    pltpu.sync_copy(lg_hbm.at[pl.ds(base, tok_per_w), :], lg_v)

    lane = lax.iota(jnp.int32, SC_LANES)
    neg_inf = jnp.full((SC_LANES,), -jnp.inf, jnp.float32)
    cnt = [jnp.zeros((SC_LANES,), jnp.float32) for _ in range(NUM_EXPERTS)]

    for j in range(n_grp):
        row = jnp.full((SC_LANES,), j * SC_LANES, jnp.int32) + lane
        v = [plsc.load_gather(lg_v, [row, jnp.full((SC_LANES,), e, jnp.int32)])
             for e in range(NUM_EXPERTS)]
        m1 = v[0]
        for e in range(1, NUM_EXPERTS):
            m1 = jnp.maximum(m1, v[e])
        i1 = jnp.full((SC_LANES,), NUM_EXPERTS, jnp.int32)
        for e in reversed(range(NUM_EXPERTS)):
            i1 = jnp.where(v[e] == m1, jnp.full((SC_LANES,), e, jnp.int32), i1)
        l2 = [jnp.where(i1 == e, neg_inf, v[e]) for e in range(NUM_EXPERTS)]
        m2 = l2[0]
        for e in range(1, NUM_EXPERTS):
            m2 = jnp.maximum(m2, l2[e])
        i2 = jnp.full((SC_LANES,), NUM_EXPERTS, jnp.int32)
        for e in reversed(range(NUM_EXPERTS)):
            i2 = jnp.where(l2[e] == m2, jnp.full((SC_LANES,), e, jnp.int32), i2)
        zero = jnp.zeros((SC_LANES,), jnp.float32)
        one = jnp.full((SC_LANES,), 1.0, jnp.float32)
        for e in range(NUM_EXPERTS):
            cnt[e] = (cnt[e] + jnp.where(i1 == e, one, zero)
                      + jnp.where(i2 == e, one, zero))
        i1_v[pl.ds(j * SC_LANES, SC_LANES)] = i1
        i2_v[pl.ds(j * SC_LANES, SC_LANES)] = i2

    for e in range(NUM_EXPERTS):
        cnt_v[e, :] = cnt[e]
    pltpu.sync_copy(i1_v, idxt_hbm.at[0, pl.ds(base, tok_per_w)])
    pltpu.sync_copy(i2_v, idxt_hbm.at[1, pl.ds(base, tok_per_w)])
    pltpu.sync_copy(cnt_v, cnt_hbm.at[wid])


def _stage3_body(h_ref, w_ref, w2_ref, b2_ref, out_ref):
    w = w_ref[...]
    exp_mat = jnp.where(
        lax.broadcasted_iota(jnp.int32, (NUM_EXPERTS, EF), 1) // D_FFN
        == lax.broadcasted_iota(jnp.int32, (NUM_EXPERTS, EF), 0),
        1.0, 0.0)
    w_exp = jnp.dot(w, exp_mat, preferred_element_type=jnp.float32)
    hw = (h_ref[...].astype(jnp.float32) * w_exp).astype(jnp.bfloat16)
    out = jnp.dot(hw, w2_ref[...], preferred_element_type=jnp.float32)
    out = out + jnp.dot(w, b2_ref[...], preferred_element_type=jnp.float32)
    out_ref[...] = out


@jax.jit
def kernel(x, task_id, task_emb, gate_W, gate_b, W1, b1, W2, b2):
    B, S, D = x.shape
    T = B * S
    nblk = T // TOK_BLK
    tok_per_w = T // SC_WORKERS

    x2d = x.reshape(T, D)
    w1b = W1.astype(jnp.bfloat16)
    b1f = b1.reshape(1, EF)
    gbr = gate_b.reshape(1, NUM_EXPERTS)
    w2a = W2.reshape(EF, D).astype(jnp.bfloat16)
    tid32 = task_id.astype(jnp.int32)

    h, logits, w_tok = pl.pallas_call(
        _stage12_body,
        grid=(2 * nblk,),
        in_specs=[
            pl.BlockSpec(memory_space=pltpu.MemorySpace.SMEM),
            pl.BlockSpec((TOK_BLK, D),
                         lambda i: (jnp.where(i < nblk, i, i - nblk), 0)),
            pl.BlockSpec((NUM_EXPERTS, D, D_FFN), lambda i: (0, 0, 0)),
            pl.BlockSpec((1, EF), lambda i: (0, 0)),
            pl.BlockSpec((D + D_TASK_EMBED, NUM_EXPERTS), lambda i: (0, 0)),
            pl.BlockSpec((1, NUM_EXPERTS), lambda i: (0, 0)),
            pl.BlockSpec((NUM_TASKS, D_TASK_EMBED), lambda i: (0, 0)),
        ],
        out_specs=[
            pl.BlockSpec((TOK_BLK, EF), lambda i: (jnp.minimum(i, nblk - 1), 0)),
            pl.BlockSpec((TOK_BLK, NUM_EXPERTS),
                         lambda i: (jnp.maximum(i - nblk, 0), 0)),
            pl.BlockSpec((TOK_BLK, NUM_EXPERTS),
                         lambda i: (jnp.maximum(i - nblk, 0), 0)),
        ],
        out_shape=[
            jax.ShapeDtypeStruct((T, EF), jnp.bfloat16),
            jax.ShapeDtypeStruct((T, NUM_EXPERTS), jnp.float32),
            jax.ShapeDtypeStruct((T, NUM_EXPERTS), jnp.float32),
        ],
        scratch_shapes=[pltpu.SMEM((2, B), jnp.float32)],
    )(tid32, x2d, w1b, b1f, gate_W, gbr, task_emb)

    mesh = plsc.VectorSubcoreMesh(core_axis_name="c", subcore_axis_name="s")
    idxt, cnt = pl.kernel(
        _route_body,
        out_type=[
            jax.ShapeDtypeStruct((TOP_K, T), jnp.int32),
            jax.ShapeDtypeStruct((SC_WORKERS, NUM_EXPERTS, SC_LANES),
                                 jnp.float32),
        ],
        mesh=mesh,
        compiler_params=pltpu.CompilerParams(needs_layout_passes=False),
        scratch_types=[
            pltpu.VMEM((tok_per_w, NUM_EXPERTS), jnp.float32),
            pltpu.VMEM((tok_per_w,), jnp.int32),
            pltpu.VMEM((tok_per_w,), jnp.int32),
            pltpu.VMEM((NUM_EXPERTS, SC_LANES), jnp.float32),
        ],
    )(logits)

    out = pl.pallas_call(
        _stage3_body,
        grid=(nblk,),
        in_specs=[
            pl.BlockSpec((TOK_BLK, EF), lambda i: (i, 0)),
            pl.BlockSpec((TOK_BLK, NUM_EXPERTS), lambda i: (i, 0)),
            pl.BlockSpec((EF, D), lambda i: (0, 0)),
            pl.BlockSpec((NUM_EXPERTS, D), lambda i: (0, 0)),
        ],
        out_specs=pl.BlockSpec((TOK_BLK, D), lambda i: (i, 0)),
        out_shape=jax.ShapeDtypeStruct((T, D), jnp.float32),
    )(h, w_tok, w2a, b2)

    final_output = out.reshape(B, S, D)
    topk_idx = jnp.transpose(idxt, (1, 0)).reshape(B, S, TOP_K)

    counts = jnp.sum(cnt, axis=(0, 2))
    usage_mean = jnp.mean(counts) + 1e-6
    usage_std = jnp.std(counts, ddof=1)
    lb_loss = (usage_std / usage_mean) ** 2
    return (final_output, lb_loss, topk_idx)


# speedup vs baseline: 1.3783x; 1.3783x over previous
import jax
import jax.numpy as jnp
from jax import lax
from jax.experimental import pallas as pl
from jax.experimental.pallas import tpu as pltpu
from jax.experimental.pallas import tpu_sc as plsc

D_MODEL = 2048
NUM_EXPERTS = 8
TOP_K = 2
NUM_TASKS = 64
D_TASK_EMBED = 64
D_FFN = 32
EF = NUM_EXPERTS * D_FFN

TOK_BLK = 512

SC_CORES = 2
SC_SUBCORES = 16
SC_WORKERS = SC_CORES * SC_SUBCORES
SC_LANES = 16


def _stage12_body(tid_ref, x_ref, w1_ref, b1_ref, gw_ref, gb_ref, temb_ref,
                  h_ref, lg_ref, w_ref, stats_ref):
    i = pl.program_id(0)
    nb = pl.num_programs(0) // 2
    blk_per_b = nb // stats_ref.shape[1]
    n_elem = jnp.float32(blk_per_b * TOK_BLK * D_MODEL)
    x = x_ref[...]

    @pl.when(i == 0)
    def _init():
        for b in range(stats_ref.shape[1]):
            stats_ref[0, b] = 0.0
            stats_ref[1, b] = 0.0

    @pl.when(i < nb)
    def _phase_a():
        b = i // blk_per_b
        stats_ref[0, b] += jnp.sum(x)
        stats_ref[1, b] += jnp.sum(x * x)
        h = jnp.dot(x.astype(jnp.bfloat16), w1_ref[...],
                    preferred_element_type=jnp.float32)
        h = h + b1_ref[...]
        h_ref[...] = (h * jax.nn.sigmoid(h)).astype(jnp.bfloat16)

    @pl.when(i >= nb)
    def _phase_b():
        b = (i - nb) // blk_per_b
        mean = stats_ref[0, b] / n_elem
        var = stats_ref[1, b] / n_elem - mean * mean
        std = jnp.sqrt(var + 1e-5)
        xn = (x - mean) / std
        tid = tid_ref[b]
        tb = temb_ref[pl.ds(tid, 1), :]
        te_row = jnp.dot(tb.astype(jnp.bfloat16),
                         gw_ref[pl.ds(D_MODEL, D_TASK_EMBED), :].astype(jnp.bfloat16),
                         preferred_element_type=jnp.float32) + gb_ref[...]
        logits = jnp.dot(xn.astype(jnp.bfloat16),
                         gw_ref[pl.ds(0, D_MODEL), :].astype(jnp.bfloat16),
                         preferred_element_type=jnp.float32) + te_row
        lg_ref[...] = logits
        ii = lax.broadcasted_iota(jnp.int32, logits.shape, 1)
        m1 = jnp.max(logits, axis=1, keepdims=True)
        i1 = jnp.min(jnp.where(logits == m1, ii, NUM_EXPERTS), axis=1,
                     keepdims=True)
        l2 = jnp.where(ii == i1, -jnp.inf, logits)
        m2 = jnp.max(l2, axis=1, keepdims=True)
        i2 = jnp.min(jnp.where(l2 == m2, ii, NUM_EXPERTS), axis=1,
                     keepdims=True)
        ed = jnp.exp(m2 - m1)
        p1 = 1.0 / (1.0 + ed)
        p2 = ed / (1.0 + ed)
        w_ref[...] = jnp.where(ii == i1, p1, 0.0) + jnp.where(ii == i2, p2, 0.0)


def _route_body(lg_hbm, idxt_hbm, cnt_hbm, lg_v, i1_v, i2_v, cnt_v):
    tok_per_w = lg_hbm.shape[0] // SC_WORKERS
    n_grp = tok_per_w // SC_LANES
    wid = lax.axis_index("s") * SC_CORES + lax.axis_index("c")
    base = wid * tok_per_w
    pltpu.sync_copy(lg_hbm.at[pl.ds(base, tok_per_w), :], lg_v)

    lane = lax.iota(jnp.int32, SC_LANES)
    neg_inf = jnp.full((SC_LANES,), -jnp.inf, jnp.float32)
    cnt = [jnp.zeros((SC_LANES,), jnp.float32) for _ in range(NUM_EXPERTS)]

    for j in range(n_grp):
        row = jnp.full((SC_LANES,), j * SC_LANES, jnp.int32) + lane
        v = [plsc.load_gather(lg_v, [row, jnp.full((SC_LANES,), e, jnp.int32)])
             for e in range(NUM_EXPERTS)]
        m1 = v[0]
        for e in range(1, NUM_EXPERTS):
            m1 = jnp.maximum(m1, v[e])
        i1 = jnp.full((SC_LANES,), NUM_EXPERTS, jnp.int32)
        for e in reversed(range(NUM_EXPERTS)):
            i1 = jnp.where(v[e] == m1, jnp.full((SC_LANES,), e, jnp.int32), i1)
        l2 = [jnp.where(i1 == e, neg_inf, v[e]) for e in range(NUM_EXPERTS)]
        m2 = l2[0]
        for e in range(1, NUM_EXPERTS):
            m2 = jnp.maximum(m2, l2[e])
        i2 = jnp.full((SC_LANES,), NUM_EXPERTS, jnp.int32)
        for e in reversed(range(NUM_EXPERTS)):
            i2 = jnp.where(l2[e] == m2, jnp.full((SC_LANES,), e, jnp.int32), i2)
        zero = jnp.zeros((SC_LANES,), jnp.float32)
        one = jnp.full((SC_LANES,), 1.0, jnp.float32)
        for e in range(NUM_EXPERTS):
            cnt[e] = (cnt[e] + jnp.where(i1 == e, one, zero)
                      + jnp.where(i2 == e, one, zero))
        i1_v[pl.ds(j * SC_LANES, SC_LANES)] = i1
        i2_v[pl.ds(j * SC_LANES, SC_LANES)] = i2

    for e in range(NUM_EXPERTS):
        cnt_v[e, :] = cnt[e]
    pltpu.sync_copy(i1_v, idxt_hbm.at[0, pl.ds(base, tok_per_w)])
    pltpu.sync_copy(i2_v, idxt_hbm.at[1, pl.ds(base, tok_per_w)])
    pltpu.sync_copy(cnt_v, cnt_hbm.at[wid])


def _stage3_body(h_ref, w_ref, w2_ref, b2_ref, out_ref):
    w = w_ref[...]
    exp_mat = jnp.where(
        lax.broadcasted_iota(jnp.int32, (NUM_EXPERTS, EF), 1) // D_FFN
        == lax.broadcasted_iota(jnp.int32, (NUM_EXPERTS, EF), 0),
        1.0, 0.0)
    w_exp = jnp.dot(w, exp_mat, preferred_element_type=jnp.float32)
    hw = (h_ref[...].astype(jnp.float32) * w_exp).astype(jnp.bfloat16)
    out = jnp.dot(hw, w2_ref[...], preferred_element_type=jnp.float32)
    out = out + jnp.dot(w, b2_ref[...], preferred_element_type=jnp.float32)
    out_ref[...] = out


@jax.jit
def kernel(x, task_id, task_emb, gate_W, gate_b, W1, b1, W2, b2):
    B, S, D = x.shape
    T = B * S
    nblk = T // TOK_BLK
    tok_per_w = T // SC_WORKERS

    x2d = x.reshape(T, D)
    w1a = W1.astype(jnp.bfloat16).transpose(1, 0, 2).reshape(D, EF)
    b1f = b1.reshape(1, EF)
    gbr = gate_b.reshape(1, NUM_EXPERTS)
    w2a = W2.reshape(EF, D).astype(jnp.bfloat16)
    tid32 = task_id.astype(jnp.int32)

    h, logits, w_tok = pl.pallas_call(
        _stage12_body,
        grid=(2 * nblk,),
        in_specs=[
            pl.BlockSpec(memory_space=pltpu.MemorySpace.SMEM),
            pl.BlockSpec((TOK_BLK, D),
                         lambda i: (jnp.where(i < nblk, i, i - nblk), 0)),
            pl.BlockSpec((D, EF), lambda i: (0, 0)),
            pl.BlockSpec((1, EF), lambda i: (0, 0)),
            pl.BlockSpec((D + D_TASK_EMBED, NUM_EXPERTS), lambda i: (0, 0)),
            pl.BlockSpec((1, NUM_EXPERTS), lambda i: (0, 0)),
            pl.BlockSpec((NUM_TASKS, D_TASK_EMBED), lambda i: (0, 0)),
        ],
        out_specs=[
            pl.BlockSpec((TOK_BLK, EF), lambda i: (jnp.minimum(i, nblk - 1), 0)),
            pl.BlockSpec((TOK_BLK, NUM_EXPERTS),
                         lambda i: (jnp.maximum(i - nblk, 0), 0)),
            pl.BlockSpec((TOK_BLK, NUM_EXPERTS),
                         lambda i: (jnp.maximum(i - nblk, 0), 0)),
        ],
        out_shape=[
            jax.ShapeDtypeStruct((T, EF), jnp.bfloat16),
            jax.ShapeDtypeStruct((T, NUM_EXPERTS), jnp.float32),
            jax.ShapeDtypeStruct((T, NUM_EXPERTS), jnp.float32),
        ],
        scratch_shapes=[pltpu.SMEM((2, B), jnp.float32)],
    )(tid32, x2d, w1a, b1f, gate_W, gbr, task_emb)

    mesh = plsc.VectorSubcoreMesh(core_axis_name="c", subcore_axis_name="s")
    idxt, cnt = pl.kernel(
        _route_body,
        out_type=[
            jax.ShapeDtypeStruct((TOP_K, T), jnp.int32),
            jax.ShapeDtypeStruct((SC_WORKERS, NUM_EXPERTS, SC_LANES),
                                 jnp.float32),
        ],
        mesh=mesh,
        compiler_params=pltpu.CompilerParams(needs_layout_passes=False),
        scratch_types=[
            pltpu.VMEM((tok_per_w, NUM_EXPERTS), jnp.float32),
            pltpu.VMEM((tok_per_w,), jnp.int32),
            pltpu.VMEM((tok_per_w,), jnp.int32),
            pltpu.VMEM((NUM_EXPERTS, SC_LANES), jnp.float32),
        ],
    )(logits)

    out = pl.pallas_call(
        _stage3_body,
        grid=(nblk,),
        in_specs=[
            pl.BlockSpec((TOK_BLK, EF), lambda i: (i, 0)),
            pl.BlockSpec((TOK_BLK, NUM_EXPERTS), lambda i: (i, 0)),
            pl.BlockSpec((EF, D), lambda i: (0, 0)),
            pl.BlockSpec((NUM_EXPERTS, D), lambda i: (0, 0)),
        ],
        out_specs=pl.BlockSpec((TOK_BLK, D), lambda i: (i, 0)),
        out_shape=jax.ShapeDtypeStruct((T, D), jnp.float32),
    )(h, w_tok, w2a, b2)

    final_output = out.reshape(B, S, D)
    topk_idx = jnp.transpose(idxt, (1, 0)).reshape(B, S, TOP_K)

    counts = jnp.sum(cnt, axis=(0, 2))
    usage_mean = jnp.mean(counts) + 1e-6
    usage_std = jnp.std(counts, ddof=1)
    lb_loss = (usage_std / usage_mean) ** 2
    return (final_output, lb_loss, topk_idx)
